# baseline (speedup 1.0000x reference)
import functools

import jax
import jax.numpy as jnp
from jax.experimental import pallas as pl
from jax.experimental.pallas import tpu as pltpu

BM = 400
BK = 4096


def _fgc_kernel(n_valid, a_ref, b_ref, f_ref, wn_ref, bn_ref, we_ref, be_ref,
                o_ref, acc_n, acc_e):
    k = pl.program_id(1)
    nk = pl.num_programs(1)

    @pl.when(k == 0)
    def _():
        acc_n[...] = jnp.zeros_like(acc_n)
        acc_e[...] = jnp.zeros_like(acc_e)

    f = f_ref[pl.ds(k * BK, BK), :].astype(jnp.bfloat16)

    @pl.when(k == nk - 1)
    def _():
        col = k * BK + jax.lax.broadcasted_iota(jnp.int32, (BM, BK), 1)
        valid = col < n_valid
        am = jnp.where(valid, a_ref[...], 0.0).astype(jnp.bfloat16)
        acc_n[...] += jax.lax.dot(am, f, preferred_element_type=jnp.float32)
        bm = jnp.where(valid, b_ref[...], 0.0).astype(jnp.bfloat16)
        acc_e[...] += jax.lax.dot(bm, f, preferred_element_type=jnp.float32)

    @pl.when(k < nk - 1)
    def _():
        a = a_ref[...].astype(jnp.bfloat16)
        acc_n[...] += jax.lax.dot(a, f, preferred_element_type=jnp.float32)
        b = b_ref[...].astype(jnp.bfloat16)
        acc_e[...] += jax.lax.dot(b, f, preferred_element_type=jnp.float32)

    @pl.when(k == nk - 1)
    def _():
        no = jax.lax.dot(acc_n[...], wn_ref[...],
                         preferred_element_type=jnp.float32) + bn_ref[...]
        eo = jax.lax.dot(acc_e[...], we_ref[...],
                         preferred_element_type=jnp.float32) + be_ref[...]
        o_ref[...] = jnp.maximum(no, 0.0) + eo


@jax.jit
def kernel(feats, node_adj, edge_adj, node_weight, node_bias, edge_weight,
           edge_bias):
    n, d_in = feats.shape
    d_out = node_weight.shape[1]
    nk = pl.cdiv(n, BK)
    grid = (n // BM, nk)
    feats_p = jnp.pad(feats, ((0, nk * BK - n), (0, 0)))

    out = pl.pallas_call(
        functools.partial(_fgc_kernel, n),
        grid=grid,
        in_specs=[
            pl.BlockSpec((BM, BK), lambda i, k: (i, k)),
            pl.BlockSpec((BM, BK), lambda i, k: (i, k)),
            pl.BlockSpec((nk * BK, d_in), lambda i, k: (0, 0)),
            pl.BlockSpec((d_in, d_out), lambda i, k: (0, 0)),
            pl.BlockSpec((1, d_out), lambda i, k: (0, 0)),
            pl.BlockSpec((d_in, d_out), lambda i, k: (0, 0)),
            pl.BlockSpec((1, d_out), lambda i, k: (0, 0)),
        ],
        out_specs=pl.BlockSpec((BM, d_out), lambda i, k: (i, 0)),
        out_shape=jax.ShapeDtypeStruct((n, d_out), jnp.float32),
        scratch_shapes=[
            pltpu.VMEM((BM, d_out), jnp.float32),
            pltpu.VMEM((BM, d_out), jnp.float32),
        ],
        compiler_params=pltpu.CompilerParams(
            dimension_semantics=("parallel", "arbitrary"),
        ),
    )(node_adj, edge_adj, feats_p, node_weight, node_bias.reshape(1, d_out),
      edge_weight, edge_bias.reshape(1, d_out))
    return out


# speedup vs baseline: 1.1237x; 1.1237x over previous
import functools

import jax
import jax.numpy as jnp
from jax.experimental import pallas as pl
from jax.experimental.pallas import tpu as pltpu

BM = 1000
BK = 2560


def _fgc_kernel(n_valid, a_ref, b_ref, f_ref, wn_ref, bn_ref, we_ref, be_ref,
                o_ref, acc_n, acc_e):
    k = pl.program_id(1)
    nk = pl.num_programs(1)

    @pl.when(k == 0)
    def _():
        acc_n[...] = jnp.zeros_like(acc_n)
        acc_e[...] = jnp.zeros_like(acc_e)

    f = f_ref[pl.ds(k * BK, BK), :].astype(jnp.bfloat16)

    @pl.when(k == nk - 1)
    def _():
        col = k * BK + jax.lax.broadcasted_iota(jnp.int32, (BM, BK), 1)
        valid = col < n_valid
        am = jnp.where(valid, a_ref[...], 0.0).astype(jnp.bfloat16)
        acc_n[...] += jax.lax.dot(am, f, preferred_element_type=jnp.float32)
        bm = jnp.where(valid, b_ref[...], 0.0).astype(jnp.bfloat16)
        acc_e[...] += jax.lax.dot(bm, f, preferred_element_type=jnp.float32)

    @pl.when(k < nk - 1)
    def _():
        a = a_ref[...].astype(jnp.bfloat16)
        acc_n[...] += jax.lax.dot(a, f, preferred_element_type=jnp.float32)
        b = b_ref[...].astype(jnp.bfloat16)
        acc_e[...] += jax.lax.dot(b, f, preferred_element_type=jnp.float32)

    @pl.when(k == nk - 1)
    def _():
        no = jax.lax.dot(acc_n[...], wn_ref[...],
                         preferred_element_type=jnp.float32) + bn_ref[...]
        eo = jax.lax.dot(acc_e[...], we_ref[...],
                         preferred_element_type=jnp.float32) + be_ref[...]
        o_ref[...] = jnp.maximum(no, 0.0) + eo


@jax.jit
def kernel(feats, node_adj, edge_adj, node_weight, node_bias, edge_weight,
           edge_bias):
    n, d_in = feats.shape
    d_out = node_weight.shape[1]
    nk = pl.cdiv(n, BK)
    grid = (n // BM, nk)
    feats_p = jnp.pad(feats, ((0, nk * BK - n), (0, 0)))

    out = pl.pallas_call(
        functools.partial(_fgc_kernel, n),
        grid=grid,
        in_specs=[
            pl.BlockSpec((BM, BK), lambda i, k: (i, k)),
            pl.BlockSpec((BM, BK), lambda i, k: (i, k)),
            pl.BlockSpec((nk * BK, d_in), lambda i, k: (0, 0)),
            pl.BlockSpec((d_in, d_out), lambda i, k: (0, 0)),
            pl.BlockSpec((1, d_out), lambda i, k: (0, 0)),
            pl.BlockSpec((d_in, d_out), lambda i, k: (0, 0)),
            pl.BlockSpec((1, d_out), lambda i, k: (0, 0)),
        ],
        out_specs=pl.BlockSpec((BM, d_out), lambda i, k: (i, 0)),
        out_shape=jax.ShapeDtypeStruct((n, d_out), jnp.float32),
        scratch_shapes=[
            pltpu.VMEM((BM, d_out), jnp.float32),
            pltpu.VMEM((BM, d_out), jnp.float32),
        ],
        compiler_params=pltpu.CompilerParams(
            dimension_semantics=("parallel", "arbitrary"),
        ),
    )(node_adj, edge_adj, feats_p, node_weight, node_bias.reshape(1, d_out),
      edge_weight, edge_bias.reshape(1, d_out))
    return out


# speedup vs baseline: 1.1342x; 1.0094x over previous
import functools

import jax
import jax.numpy as jnp
from jax.experimental import pallas as pl
from jax.experimental.pallas import tpu as pltpu

BM = 1000
BK = 2048


def _fgc_kernel(n_valid, n_pad, a_ref, b_ref, f_hbm, wn_ref, bn_ref, we_ref,
                be_ref, o_ref, acc_n, acc_e, f_vmem, f_sem):
    i = pl.program_id(0)
    k = pl.program_id(1)
    nk = pl.num_programs(1)

    @pl.when(jnp.logical_and(i == 0, k == 0))
    def _():
        if n_pad > n_valid:
            f_vmem[pl.ds(n_valid, n_pad - n_valid), :] = jnp.zeros(
                (n_pad - n_valid, f_vmem.shape[1]), jnp.float32)
        cp = pltpu.make_async_copy(f_hbm, f_vmem.at[pl.ds(0, n_valid), :],
                                   f_sem)
        cp.start()
        cp.wait()

    @pl.when(k == 0)
    def _():
        acc_n[...] = jnp.zeros_like(acc_n)
        acc_e[...] = jnp.zeros_like(acc_e)

    f = f_vmem[pl.ds(k * BK, BK), :].astype(jnp.bfloat16)

    @pl.when(k == nk - 1)
    def _():
        col = k * BK + jax.lax.broadcasted_iota(jnp.int32, (BM, BK), 1)
        valid = col < n_valid
        am = jnp.where(valid, a_ref[...], 0.0).astype(jnp.bfloat16)
        acc_n[...] += jax.lax.dot(am, f, preferred_element_type=jnp.float32)
        bm = jnp.where(valid, b_ref[...], 0.0).astype(jnp.bfloat16)
        acc_e[...] += jax.lax.dot(bm, f, preferred_element_type=jnp.float32)

    @pl.when(k < nk - 1)
    def _():
        a = a_ref[...].astype(jnp.bfloat16)
        acc_n[...] += jax.lax.dot(a, f, preferred_element_type=jnp.float32)
        b = b_ref[...].astype(jnp.bfloat16)
        acc_e[...] += jax.lax.dot(b, f, preferred_element_type=jnp.float32)

    @pl.when(k == nk - 1)
    def _():
        no = jax.lax.dot(acc_n[...], wn_ref[...],
                         preferred_element_type=jnp.float32) + bn_ref[...]
        eo = jax.lax.dot(acc_e[...], we_ref[...],
                         preferred_element_type=jnp.float32) + be_ref[...]
        o_ref[...] = jnp.maximum(no, 0.0) + eo


@jax.jit
def kernel(feats, node_adj, edge_adj, node_weight, node_bias, edge_weight,
           edge_bias):
    n, d_in = feats.shape
    d_out = node_weight.shape[1]
    nk = pl.cdiv(n, BK)
    n_pad = nk * BK
    grid = (n // BM, nk)

    out = pl.pallas_call(
        functools.partial(_fgc_kernel, n, n_pad),
        grid=grid,
        in_specs=[
            pl.BlockSpec((BM, BK), lambda i, k: (i, k)),
            pl.BlockSpec((BM, BK), lambda i, k: (i, k)),
            pl.BlockSpec(memory_space=pltpu.MemorySpace.HBM),
            pl.BlockSpec((d_in, d_out), lambda i, k: (0, 0)),
            pl.BlockSpec((1, d_out), lambda i, k: (0, 0)),
            pl.BlockSpec((d_in, d_out), lambda i, k: (0, 0)),
            pl.BlockSpec((1, d_out), lambda i, k: (0, 0)),
        ],
        out_specs=pl.BlockSpec((BM, d_out), lambda i, k: (i, 0)),
        out_shape=jax.ShapeDtypeStruct((n, d_out), jnp.float32),
        scratch_shapes=[
            pltpu.VMEM((BM, d_out), jnp.float32),
            pltpu.VMEM((BM, d_out), jnp.float32),
            pltpu.VMEM((n_pad, d_in), jnp.float32),
            pltpu.SemaphoreType.DMA,
        ],
        compiler_params=pltpu.CompilerParams(
            dimension_semantics=("parallel", "arbitrary"),
        ),
    )(node_adj, edge_adj, feats, node_weight, node_bias.reshape(1, d_out),
      edge_weight, edge_bias.reshape(1, d_out))
    return out
